# baseline (speedup 1.0000x reference)
import functools

import jax
import jax.numpy as jnp
from jax import lax
from jax.experimental import pallas as pl
from jax.experimental.pallas import tpu as pltpu
from jax.experimental.pallas import tpu_sc as plsc

N = 10000
E = 160000
D = 256
U = 256
NC = 2
NS = 16
L = 16
NP = 10240
EPAD = 163840
CH = 64
HALF = D // 2

_sc_mesh = plsc.VectorSubcoreMesh(core_axis_name="c", subcore_axis_name="s")


@functools.partial(
    pl.kernel,
    out_type=jax.ShapeDtypeStruct((NP,), jnp.float32),
    mesh=_sc_mesh,
    scratch_types=[
        pltpu.VMEM_SHARED((NP,), jnp.float32),
        pltpu.VMEM((CH,), jnp.int32),
        pltpu.VMEM((CH,), jnp.float32),
        pltpu.VMEM((NP // NS,), jnp.float32),
    ],
)
def _degree_kernel(dst_hbm, deg_hbm, deg_sh, dstbuf, ones, zbuf):
    c = lax.axis_index("c")
    s = lax.axis_index("s")
    tid = c * NS + s
    ept = EPAD // (NC * NS)
    zslice = NP // NS

    def fill(i, _):
        zbuf[pl.ds(i * L, L)] = jnp.zeros((L,), jnp.float32)
        return 0

    lax.fori_loop(0, zslice // L, fill, 0)
    for j in range(CH // L):
        ones[pl.ds(j * L, L)] = jnp.ones((L,), jnp.float32)

    pltpu.sync_copy(zbuf, deg_sh.at[pl.ds(s * zslice, zslice)])
    plsc.subcore_barrier()

    def body(g, _):
        b = tid * ept + g * CH
        pltpu.sync_copy(dst_hbm.at[pl.ds(b, CH)], dstbuf)
        pltpu.sync_copy(ones, deg_sh.at[dstbuf], add=True)
        return 0

    lax.fori_loop(0, ept // CH, body, 0)
    plsc.subcore_barrier()

    @pl.when(c == 0)
    def _():
        pltpu.sync_copy(deg_sh.at[pl.ds(s * zslice, zslice)],
                        deg_hbm.at[pl.ds(s * zslice, zslice)])


def _norm_body(x_ref, deg_ref, xs_ref, isd_ref):
    isd = lax.rsqrt(deg_ref[...])
    xn = isd * x_ref[...]
    xs_ref[0] = xn[:, :HALF]
    xs_ref[1] = xn[:, HALF:]
    isd_ref[...] = isd


def _norm_call(x, degc):
    R = 2000
    grid = (N // R,)
    return pl.pallas_call(
        _norm_body,
        grid=grid,
        in_specs=[
            pl.BlockSpec((R, D), lambda i: (i, 0)),
            pl.BlockSpec((R, 1), lambda i: (i, 0)),
        ],
        out_specs=[
            pl.BlockSpec((2, R, HALF), lambda i: (0, i, 0)),
            pl.BlockSpec((R, 1), lambda i: (i, 0)),
        ],
        out_shape=[
            jax.ShapeDtypeStruct((2, N, HALF), jnp.float32),
            jax.ShapeDtypeStruct((N, 1), jnp.float32),
        ],
    )(x, degc)


NCHT = EPAD // NS // CH
NB = 2
NI = 8


@functools.partial(
    pl.kernel,
    out_type=jax.ShapeDtypeStruct((NC * NP, HALF), jnp.float32),
    mesh=_sc_mesh,
    scratch_types=[
        pltpu.VMEM_SHARED((NP, HALF), jnp.float32),
        [pltpu.VMEM((1, CH), jnp.int32) for _ in range(NI)],
        [pltpu.VMEM((1, CH), jnp.int32) for _ in range(NI)],
        [pltpu.VMEM((CH, HALF), jnp.float32) for _ in range(NB)],
        [pltpu.SemaphoreType.DMA for _ in range(NB)],
        [pltpu.SemaphoreType.DMA for _ in range(NI)],
        [pltpu.SemaphoreType.DMA for _ in range(NI)],
        [pltpu.SemaphoreType.DMA for _ in range(NB)],
        pltpu.VMEM((8, HALF), jnp.float32),
    ],
)
def _pool_kernel(src_hbm, dst_hbm, xs_hbm, out_hbm,
                 pooled_sh, srcb, dstb, rows, gsem, psem, dsem, ssem, zrows):
    c = lax.axis_index("c")
    s = lax.axis_index("s")
    zslice = NP // NS
    off = c * N

    for i in range(8):
        for j in range(HALF // L):
            zrows[i, pl.ds(j * L, L)] = jnp.zeros((L,), jnp.float32)

    def zero(k, _):
        pltpu.sync_copy(zrows, pooled_sh.at[pl.ds(s * zslice + k * 8, 8), :])
        return 0

    lax.fori_loop(0, zslice // 8, zero, 0)
    plsc.subcore_barrier()

    def start_src(g, i):
        pltpu.async_copy(src_hbm.at[pl.ds(s * NCHT + g, 1), :], srcb[i],
                         psem[i])

    def wait_src(g, i):
        pltpu.make_async_copy(src_hbm.at[pl.ds(s * NCHT + g, 1), :],
                              srcb[i], psem[i]).wait()

    def start_dst(g, i):
        pltpu.async_copy(dst_hbm.at[pl.ds(s * NCHT + g, 1), :], dstb[i],
                         dsem[i])

    def wait_dst(g, i):
        pltpu.make_async_copy(dst_hbm.at[pl.ds(s * NCHT + g, 1), :],
                              dstb[i], dsem[i]).wait()

    def addoff(i):
        for j in range(CH // L):
            v = srcb[i][0, pl.ds(j * L, L)]
            srcb[i][0, pl.ds(j * L, L)] = v + off

    def start_gather(g, b, i):
        pltpu.async_copy(xs_hbm.at[srcb[i].at[0]], rows[b], gsem[b])

    def wait_gather(g, b, i):
        pltpu.make_async_copy(xs_hbm.at[srcb[i].at[0]], rows[b],
                              gsem[b]).wait()

    def start_scatter(g, b, i):
        pltpu.async_copy(rows[b], pooled_sh.at[dstb[i].at[0]], ssem[b],
                         add=True)

    def wait_scatter(g, b, i):
        pltpu.make_async_copy(rows[b], pooled_sh.at[dstb[i].at[0]],
                              ssem[b]).wait()

    for k in range(NI):
        start_src(k, k)
        start_dst(k, k)
    for b in range(NB):
        wait_src(b, b)
        addoff(b)
        start_gather(b, b, b)

    def body(k, _):
        for ii in range(NI):
            g = k * NI + ii
            bb = ii % NB
            wait_gather(g, bb, ii)
            wait_dst(g, ii)
            start_scatter(g, bb, ii)
            gi = g + NI
            gn = g + NB

            @pl.when(gn < NCHT)
            def _():
                wait_scatter(g, bb, ii)
                inew = (ii + NB) % NI

                @pl.when(gi < NCHT)
                def _():
                    start_src(gi, ii)
                    start_dst(gi, ii)

                wait_src(gn, inew)
                addoff(inew)
                start_gather(gn, bb, inew)
        return 0

    lax.fori_loop(0, NCHT // NI, body, 0)
    for bb in range(NB):
        g = NCHT - NB + bb
        wait_scatter(g, bb, g % NI)
    plsc.subcore_barrier()

    pltpu.sync_copy(
        pooled_sh.at[pl.ds(s * zslice, zslice), :],
        out_hbm.at[pl.ds(c * NP + s * zslice, zslice), :],
    )


def _dense_body(p_ref, isd_ref, w_ref, b_ref, o_ref):
    acc = jnp.dot(p_ref[0], w_ref[0], preferred_element_type=jnp.float32)
    acc += jnp.dot(p_ref[1], w_ref[1], preferred_element_type=jnp.float32)
    o_ref[...] = jnp.maximum(isd_ref[...] * acc + b_ref[...], 0.0)


def _dense_call(pooled3, isd, w3, b2):
    R = 2000
    grid = (N // R,)
    return pl.pallas_call(
        _dense_body,
        grid=grid,
        in_specs=[
            pl.BlockSpec((2, R, HALF), lambda i: (0, i, 0)),
            pl.BlockSpec((R, 1), lambda i: (i, 0)),
            pl.BlockSpec((2, HALF, U), lambda i: (0, 0, 0)),
            pl.BlockSpec((1, U), lambda i: (0, 0)),
        ],
        out_specs=pl.BlockSpec((R, U), lambda i: (i, 0)),
        out_shape=jax.ShapeDtypeStruct((N, U), jnp.float32),
    )(pooled3, isd, w3, b2)


@jax.jit
def kernel(x, edge_index, W, b):
    src = edge_index[0]
    dst = edge_index[1]
    pad = EPAD - E
    src_pad = jnp.concatenate([src, jnp.zeros((pad,), jnp.int32)])
    dst_pad = jnp.concatenate([dst, jnp.full((pad,), NP - 1, jnp.int32)])

    deg = _degree_kernel(dst_pad)
    degc = deg[:N].reshape(N, 1)
    xs3, isd = _norm_call(x, degc)
    pooled = _pool_kernel(src_pad.reshape(EPAD // CH, CH),
                          dst_pad.reshape(EPAD // CH, CH),
                          xs3.reshape(2 * N, HALF))
    pooled3 = pooled.reshape(NC, NP, HALF)
    out = _dense_call(pooled3, isd, W.reshape(2, HALF, U), b.reshape(1, U))
    return out


# speedup vs baseline: 1.0761x; 1.0761x over previous
import functools

import jax
import jax.numpy as jnp
from jax import lax
from jax.experimental import pallas as pl
from jax.experimental.pallas import tpu as pltpu
from jax.experimental.pallas import tpu_sc as plsc

N = 10000
E = 160000
D = 256
U = 256
NC = 2
NS = 16
L = 16
NP = 10240
EPAD = 163840
CH = 160
HALF = D // 2

_sc_mesh = plsc.VectorSubcoreMesh(core_axis_name="c", subcore_axis_name="s")


@functools.partial(
    pl.kernel,
    out_type=jax.ShapeDtypeStruct((NP,), jnp.float32),
    mesh=_sc_mesh,
    scratch_types=[
        pltpu.VMEM_SHARED((NP,), jnp.float32),
        pltpu.VMEM((CH,), jnp.int32),
        pltpu.VMEM((CH,), jnp.float32),
        pltpu.VMEM((NP // NS,), jnp.float32),
    ],
)
def _degree_kernel(dst_hbm, deg_hbm, deg_sh, dstbuf, ones, zbuf):
    c = lax.axis_index("c")
    s = lax.axis_index("s")
    tid = c * NS + s
    ept = EPAD // (NC * NS)
    zslice = NP // NS

    def fill(i, _):
        zbuf[pl.ds(i * L, L)] = jnp.zeros((L,), jnp.float32)
        return 0

    lax.fori_loop(0, zslice // L, fill, 0)
    for j in range(CH // L):
        ones[pl.ds(j * L, L)] = jnp.ones((L,), jnp.float32)

    pltpu.sync_copy(zbuf, deg_sh.at[pl.ds(s * zslice, zslice)])
    plsc.subcore_barrier()

    def body(g, _):
        b = tid * ept + g * CH
        pltpu.sync_copy(dst_hbm.at[pl.ds(b, CH)], dstbuf)
        pltpu.sync_copy(ones, deg_sh.at[dstbuf], add=True)
        return 0

    lax.fori_loop(0, ept // CH, body, 0)
    plsc.subcore_barrier()

    @pl.when(c == 0)
    def _():
        pltpu.sync_copy(deg_sh.at[pl.ds(s * zslice, zslice)],
                        deg_hbm.at[pl.ds(s * zslice, zslice)])


def _norm_body(x_ref, deg_ref, xs_ref, isd_ref):
    isd = lax.rsqrt(deg_ref[...])
    xn = isd * x_ref[...]
    xs_ref[0] = xn[:, :HALF]
    xs_ref[1] = xn[:, HALF:]
    isd_ref[...] = isd


def _norm_call(x, degc):
    R = 2000
    grid = (N // R,)
    return pl.pallas_call(
        _norm_body,
        grid=grid,
        in_specs=[
            pl.BlockSpec((R, D), lambda i: (i, 0)),
            pl.BlockSpec((R, 1), lambda i: (i, 0)),
        ],
        out_specs=[
            pl.BlockSpec((2, R, HALF), lambda i: (0, i, 0)),
            pl.BlockSpec((R, 1), lambda i: (i, 0)),
        ],
        out_shape=[
            jax.ShapeDtypeStruct((2, N, HALF), jnp.float32),
            jax.ShapeDtypeStruct((N, 1), jnp.float32),
        ],
    )(x, degc)


NCHT = EPAD // NS // CH
NB = 2
NI = 4


@functools.partial(
    pl.kernel,
    out_type=jax.ShapeDtypeStruct((NC * NP, HALF), jnp.float32),
    mesh=_sc_mesh,
    scratch_types=[
        pltpu.VMEM_SHARED((NP, HALF), jnp.float32),
        [pltpu.VMEM((1, CH), jnp.int32) for _ in range(NI)],
        [pltpu.VMEM((1, CH), jnp.int32) for _ in range(NI)],
        [pltpu.VMEM((CH, HALF), jnp.float32) for _ in range(NB)],
        [pltpu.SemaphoreType.DMA for _ in range(NB)],
        [pltpu.SemaphoreType.DMA for _ in range(NI)],
        [pltpu.SemaphoreType.DMA for _ in range(NI)],
        [pltpu.SemaphoreType.DMA for _ in range(NB)],
        pltpu.VMEM((8, HALF), jnp.float32),
    ],
)
def _pool_kernel(src_hbm, dst_hbm, xs_hbm, out_hbm,
                 pooled_sh, srcb, dstb, rows, gsem, psem, dsem, ssem, zrows):
    c = lax.axis_index("c")
    s = lax.axis_index("s")
    zslice = NP // NS
    off = c * N

    for i in range(8):
        for j in range(HALF // L):
            zrows[i, pl.ds(j * L, L)] = jnp.zeros((L,), jnp.float32)

    def zero(k, _):
        pltpu.sync_copy(zrows, pooled_sh.at[pl.ds(s * zslice + k * 8, 8), :])
        return 0

    lax.fori_loop(0, zslice // 8, zero, 0)
    plsc.subcore_barrier()

    def start_src(g, i):
        pltpu.async_copy(src_hbm.at[pl.ds(s * NCHT + g, 1), :], srcb[i],
                         psem[i])

    def wait_src(g, i):
        pltpu.make_async_copy(src_hbm.at[pl.ds(s * NCHT + g, 1), :],
                              srcb[i], psem[i]).wait()

    def start_dst(g, i):
        pltpu.async_copy(dst_hbm.at[pl.ds(s * NCHT + g, 1), :], dstb[i],
                         dsem[i])

    def wait_dst(g, i):
        pltpu.make_async_copy(dst_hbm.at[pl.ds(s * NCHT + g, 1), :],
                              dstb[i], dsem[i]).wait()

    def addoff(i):
        for j in range(CH // L):
            v = srcb[i][0, pl.ds(j * L, L)]
            srcb[i][0, pl.ds(j * L, L)] = v + off

    def start_gather(g, b, i):
        pltpu.async_copy(xs_hbm.at[srcb[i].at[0]], rows[b], gsem[b])

    def wait_gather(g, b, i):
        pltpu.make_async_copy(xs_hbm.at[srcb[i].at[0]], rows[b],
                              gsem[b]).wait()

    def start_scatter(g, b, i):
        pltpu.async_copy(rows[b], pooled_sh.at[dstb[i].at[0]], ssem[b],
                         add=True)

    def wait_scatter(g, b, i):
        pltpu.make_async_copy(rows[b], pooled_sh.at[dstb[i].at[0]],
                              ssem[b]).wait()

    for k in range(NI):
        start_src(k, k)
        start_dst(k, k)
    for b in range(NB):
        wait_src(b, b)
        addoff(b)
        start_gather(b, b, b)

    def body(k, _):
        for ii in range(NI):
            g = k * NI + ii
            bb = ii % NB
            wait_gather(g, bb, ii)
            wait_dst(g, ii)
            start_scatter(g, bb, ii)
            gi = g + NI
            gn = g + NB

            @pl.when(gn < NCHT)
            def _():
                wait_scatter(g, bb, ii)
                inew = (ii + NB) % NI

                @pl.when(gi < NCHT)
                def _():
                    start_src(gi, ii)
                    start_dst(gi, ii)

                wait_src(gn, inew)
                addoff(inew)
                start_gather(gn, bb, inew)
        return 0

    lax.fori_loop(0, NCHT // NI, body, 0)
    for bb in range(NB):
        g = NCHT - NB + bb
        wait_scatter(g, bb, g % NI)
    plsc.subcore_barrier()

    pltpu.sync_copy(
        pooled_sh.at[pl.ds(s * zslice, zslice), :],
        out_hbm.at[pl.ds(c * NP + s * zslice, zslice), :],
    )


def _dense_body(p_ref, isd_ref, w_ref, b_ref, o_ref):
    acc = jnp.dot(p_ref[0], w_ref[0], preferred_element_type=jnp.float32)
    acc += jnp.dot(p_ref[1], w_ref[1], preferred_element_type=jnp.float32)
    o_ref[...] = jnp.maximum(isd_ref[...] * acc + b_ref[...], 0.0)


def _dense_call(pooled3, isd, w3, b2):
    R = 2000
    grid = (N // R,)
    return pl.pallas_call(
        _dense_body,
        grid=grid,
        in_specs=[
            pl.BlockSpec((2, R, HALF), lambda i: (0, i, 0)),
            pl.BlockSpec((R, 1), lambda i: (i, 0)),
            pl.BlockSpec((2, HALF, U), lambda i: (0, 0, 0)),
            pl.BlockSpec((1, U), lambda i: (0, 0)),
        ],
        out_specs=pl.BlockSpec((R, U), lambda i: (i, 0)),
        out_shape=jax.ShapeDtypeStruct((N, U), jnp.float32),
    )(pooled3, isd, w3, b2)


@jax.jit
def kernel(x, edge_index, W, b):
    src = edge_index[0]
    dst = edge_index[1]
    pad = EPAD - E
    src_pad = jnp.concatenate([src, jnp.zeros((pad,), jnp.int32)])
    dst_pad = jnp.concatenate([dst, jnp.full((pad,), NP - 1, jnp.int32)])

    deg = _degree_kernel(dst_pad)
    degc = deg[:N].reshape(N, 1)
    xs3, isd = _norm_call(x, degc)
    pooled = _pool_kernel(src_pad.reshape(EPAD // CH, CH),
                          dst_pad.reshape(EPAD // CH, CH),
                          xs3.reshape(2 * N, HALF))
    pooled3 = pooled.reshape(NC, NP, HALF)
    out = _dense_call(pooled3, isd, W.reshape(2, HALF, U), b.reshape(1, U))
    return out


# speedup vs baseline: 1.0851x; 1.0084x over previous
import functools

import jax
import jax.numpy as jnp
from jax import lax
from jax.experimental import pallas as pl
from jax.experimental.pallas import tpu as pltpu
from jax.experimental.pallas import tpu_sc as plsc

N = 10000
E = 160000
D = 256
U = 256
NC = 2
NS = 16
L = 16
NP = 10240
EPAD = 163840
CH = 160
HALF = D // 2

_sc_mesh = plsc.VectorSubcoreMesh(core_axis_name="c", subcore_axis_name="s")


@functools.partial(
    pl.kernel,
    out_type=jax.ShapeDtypeStruct((NP,), jnp.float32),
    mesh=_sc_mesh,
    scratch_types=[
        pltpu.VMEM_SHARED((NP,), jnp.float32),
        pltpu.VMEM((CH,), jnp.int32),
        pltpu.VMEM((CH,), jnp.float32),
        pltpu.VMEM((NP // NS,), jnp.float32),
    ],
)
def _degree_kernel(dst_hbm, deg_hbm, deg_sh, dstbuf, ones, zbuf):
    c = lax.axis_index("c")
    s = lax.axis_index("s")
    tid = c * NS + s
    ept = EPAD // (NC * NS)
    zslice = NP // NS

    def fill(i, _):
        zbuf[pl.ds(i * L, L)] = jnp.zeros((L,), jnp.float32)
        return 0

    lax.fori_loop(0, zslice // L, fill, 0)
    for j in range(CH // L):
        ones[pl.ds(j * L, L)] = jnp.ones((L,), jnp.float32)

    pltpu.sync_copy(zbuf, deg_sh.at[pl.ds(s * zslice, zslice)])
    plsc.subcore_barrier()

    def body(g, _):
        b = tid * ept + g * CH
        pltpu.sync_copy(dst_hbm.at[pl.ds(b, CH)], dstbuf)
        pltpu.sync_copy(ones, deg_sh.at[dstbuf], add=True)
        return 0

    lax.fori_loop(0, ept // CH, body, 0)
    plsc.subcore_barrier()

    @pl.when(c == 0)
    def _():
        pltpu.sync_copy(deg_sh.at[pl.ds(s * zslice, zslice)],
                        deg_hbm.at[pl.ds(s * zslice, zslice)])


def _norm_body(x_ref, deg_ref, xs_ref, isd_ref):
    isd = lax.rsqrt(deg_ref[...])
    xn = isd * x_ref[...]
    xs_ref[0] = xn[:, :HALF]
    xs_ref[1] = xn[:, HALF:]
    isd_ref[...] = isd


def _norm_call(x, degc):
    R = 2000
    grid = (N // R,)
    return pl.pallas_call(
        _norm_body,
        grid=grid,
        in_specs=[
            pl.BlockSpec((R, D), lambda i: (i, 0)),
            pl.BlockSpec((R, 1), lambda i: (i, 0)),
        ],
        out_specs=[
            pl.BlockSpec((2, R, HALF), lambda i: (0, i, 0)),
            pl.BlockSpec((R, 1), lambda i: (i, 0)),
        ],
        out_shape=[
            jax.ShapeDtypeStruct((2, N, HALF), jnp.float32),
            jax.ShapeDtypeStruct((N, 1), jnp.float32),
        ],
    )(x, degc)


NCHT = EPAD // NS // CH
NB = 2
NI = 4


@functools.partial(
    pl.kernel,
    out_type=jax.ShapeDtypeStruct((NC * NP, HALF), jnp.float32),
    mesh=_sc_mesh,
    scratch_types=[
        pltpu.VMEM_SHARED((NP, HALF), jnp.float32),
        [pltpu.VMEM((1, CH), jnp.int32) for _ in range(NI)],
        [pltpu.VMEM((1, CH), jnp.int32) for _ in range(NI)],
        [pltpu.VMEM((CH, HALF), jnp.float32) for _ in range(NB)],
        [pltpu.SemaphoreType.DMA for _ in range(NB)],
        [pltpu.SemaphoreType.DMA for _ in range(NI)],
        [pltpu.SemaphoreType.DMA for _ in range(NI)],
        [pltpu.SemaphoreType.DMA for _ in range(NB)],
    ],
)
def _pool_kernel(src_hbm, dst_hbm, xs_hbm, out_hbm,
                 pooled_sh, srcb, dstb, rows, gsem, psem, dsem, ssem):
    c = lax.axis_index("c")
    s = lax.axis_index("s")
    zslice = NP // NS
    off = c * N

    def zfill(r, _):
        for j in range(HALF // L):
            rows[0][r, pl.ds(j * L, L)] = jnp.zeros((L,), jnp.float32)
        return 0

    lax.fori_loop(0, CH, zfill, 0)

    def zero(k, _):
        pltpu.sync_copy(rows[0],
                        pooled_sh.at[pl.ds(s * zslice + k * CH, CH), :])
        return 0

    lax.fori_loop(0, zslice // CH, zero, 0)
    plsc.subcore_barrier()

    def start_src(g, i):
        pltpu.async_copy(src_hbm.at[pl.ds(s * NCHT + g, 1), :], srcb[i],
                         psem[i])

    def wait_src(g, i):
        pltpu.make_async_copy(src_hbm.at[pl.ds(s * NCHT + g, 1), :],
                              srcb[i], psem[i]).wait()

    def start_dst(g, i):
        pltpu.async_copy(dst_hbm.at[pl.ds(s * NCHT + g, 1), :], dstb[i],
                         dsem[i])

    def wait_dst(g, i):
        pltpu.make_async_copy(dst_hbm.at[pl.ds(s * NCHT + g, 1), :],
                              dstb[i], dsem[i]).wait()

    def addoff(i):
        for j in range(CH // L):
            v = srcb[i][0, pl.ds(j * L, L)]
            srcb[i][0, pl.ds(j * L, L)] = v + off

    def start_gather(g, b, i):
        pltpu.async_copy(xs_hbm.at[srcb[i].at[0]], rows[b], gsem[b])

    def wait_gather(g, b, i):
        pltpu.make_async_copy(xs_hbm.at[srcb[i].at[0]], rows[b],
                              gsem[b]).wait()

    def start_scatter(g, b, i):
        pltpu.async_copy(rows[b], pooled_sh.at[dstb[i].at[0]], ssem[b],
                         add=True)

    def wait_scatter(g, b, i):
        pltpu.make_async_copy(rows[b], pooled_sh.at[dstb[i].at[0]],
                              ssem[b]).wait()

    for k in range(NI):
        start_src(k, k)
        start_dst(k, k)
    for b in range(NB):
        wait_src(b, b)
        addoff(b)
        start_gather(b, b, b)

    def body(k, _):
        for ii in range(NI):
            g = k * NI + ii
            bb = ii % NB
            wait_gather(g, bb, ii)
            wait_dst(g, ii)
            start_scatter(g, bb, ii)
            gi = g + NI
            gn = g + NB

            @pl.when(gn < NCHT)
            def _():
                wait_scatter(g, bb, ii)
                inew = (ii + NB) % NI

                @pl.when(gi < NCHT)
                def _():
                    start_src(gi, ii)
                    start_dst(gi, ii)

                wait_src(gn, inew)
                addoff(inew)
                start_gather(gn, bb, inew)
        return 0

    lax.fori_loop(0, NCHT // NI, body, 0)
    for bb in range(NB):
        g = NCHT - NB + bb
        wait_scatter(g, bb, g % NI)
    plsc.subcore_barrier()

    pltpu.sync_copy(
        pooled_sh.at[pl.ds(s * zslice, zslice), :],
        out_hbm.at[pl.ds(c * NP + s * zslice, zslice), :],
    )


def _dense_body(p_ref, isd_ref, w_ref, b_ref, o_ref):
    acc = jnp.dot(p_ref[0], w_ref[0], preferred_element_type=jnp.float32)
    acc += jnp.dot(p_ref[1], w_ref[1], preferred_element_type=jnp.float32)
    o_ref[...] = jnp.maximum(isd_ref[...] * acc + b_ref[...], 0.0)


def _dense_call(pooled3, isd, w3, b2):
    R = 2000
    grid = (N // R,)
    return pl.pallas_call(
        _dense_body,
        grid=grid,
        in_specs=[
            pl.BlockSpec((2, R, HALF), lambda i: (0, i, 0)),
            pl.BlockSpec((R, 1), lambda i: (i, 0)),
            pl.BlockSpec((2, HALF, U), lambda i: (0, 0, 0)),
            pl.BlockSpec((1, U), lambda i: (0, 0)),
        ],
        out_specs=pl.BlockSpec((R, U), lambda i: (i, 0)),
        out_shape=jax.ShapeDtypeStruct((N, U), jnp.float32),
    )(pooled3, isd, w3, b2)


@jax.jit
def kernel(x, edge_index, W, b):
    src = edge_index[0]
    dst = edge_index[1]
    pad = EPAD - E
    src_pad = jnp.concatenate([src, jnp.zeros((pad,), jnp.int32)])
    dst_pad = jnp.concatenate([dst, jnp.full((pad,), NP - 1, jnp.int32)])

    deg = _degree_kernel(dst_pad)
    degc = deg[:N].reshape(N, 1)
    xs3, isd = _norm_call(x, degc)
    pooled = _pool_kernel(src_pad.reshape(EPAD // CH, CH),
                          dst_pad.reshape(EPAD // CH, CH),
                          xs3.reshape(2 * N, HALF))
    pooled3 = pooled.reshape(NC, NP, HALF)
    out = _dense_call(pooled3, isd, W.reshape(2, HALF, U), b.reshape(1, U))
    return out


# speedup vs baseline: 1.1483x; 1.0582x over previous
import functools

import jax
import jax.numpy as jnp
from jax import lax
from jax.experimental import pallas as pl
from jax.experimental.pallas import tpu as pltpu
from jax.experimental.pallas import tpu_sc as plsc

N = 10000
E = 160000
D = 256
U = 256
NC = 2
NS = 16
L = 16
NP = 10240
EPAD = 163840
CH = 160
HALF = D // 2

_sc_mesh = plsc.VectorSubcoreMesh(core_axis_name="c", subcore_axis_name="s")


@functools.partial(
    pl.kernel,
    out_type=jax.ShapeDtypeStruct((NC, NP), jnp.float32),
    mesh=_sc_mesh,
    scratch_types=[
        pltpu.VMEM_SHARED((NP,), jnp.float32),
        pltpu.VMEM((EPAD // (NC * NS) // 128, 128), jnp.int32),
        pltpu.VMEM((128,), jnp.float32),
        pltpu.VMEM((NP // NS,), jnp.float32),
    ],
)
def _degree_kernel(dst_hbm, deg_hbm, deg_sh, dstbuf, ones, zbuf):
    c = lax.axis_index("c")
    s = lax.axis_index("s")
    tid = c * NS + s
    ept = EPAD // (NC * NS)
    nch = ept // 128
    zslice = NP // NS

    def fill(i, _):
        zbuf[pl.ds(i * L, L)] = jnp.zeros((L,), jnp.float32)
        return 0

    lax.fori_loop(0, zslice // L, fill, 0)
    for i in range(128 // L):
        ones[pl.ds(i * L, L)] = jnp.ones((L,), jnp.float32)

    pltpu.sync_copy(zbuf, deg_sh.at[pl.ds(s * zslice, zslice)])
    pltpu.sync_copy(dst_hbm.at[pl.ds(tid * nch, nch), :], dstbuf)
    plsc.subcore_barrier()

    def body(g, _):
        pltpu.sync_copy(ones, deg_sh.at[dstbuf.at[g]], add=True)
        return 0

    lax.fori_loop(0, nch, body, 0)
    plsc.subcore_barrier()

    pltpu.sync_copy(deg_sh.at[pl.ds(s * zslice, zslice)],
                    deg_hbm.at[c, pl.ds(s * zslice, zslice)])


def _norm_body(x_ref, deg0_ref, deg1_ref, xs_ref, isd_ref):
    isd = lax.rsqrt(deg0_ref[...] + deg1_ref[...])
    xn = isd * x_ref[...]
    xs_ref[0] = xn[:, :HALF]
    xs_ref[1] = xn[:, HALF:]
    isd_ref[...] = isd


def _norm_call(x, degc0, degc1):
    R = 2000
    grid = (N // R,)
    return pl.pallas_call(
        _norm_body,
        grid=grid,
        in_specs=[
            pl.BlockSpec((R, D), lambda i: (i, 0)),
            pl.BlockSpec((R, 1), lambda i: (i, 0)),
            pl.BlockSpec((R, 1), lambda i: (i, 0)),
        ],
        out_specs=[
            pl.BlockSpec((2, R, HALF), lambda i: (0, i, 0)),
            pl.BlockSpec((R, 1), lambda i: (i, 0)),
        ],
        out_shape=[
            jax.ShapeDtypeStruct((2, N, HALF), jnp.float32),
            jax.ShapeDtypeStruct((N, 1), jnp.float32),
        ],
    )(x, degc0, degc1)


NCHT = EPAD // NS // CH
NB = 2
NI = 4


@functools.partial(
    pl.kernel,
    out_type=jax.ShapeDtypeStruct((NC * NP, HALF), jnp.float32),
    mesh=_sc_mesh,
    scratch_types=[
        pltpu.VMEM_SHARED((NP, HALF), jnp.float32),
        [pltpu.VMEM((1, CH), jnp.int32) for _ in range(NI)],
        [pltpu.VMEM((1, CH), jnp.int32) for _ in range(NI)],
        [pltpu.VMEM((CH, HALF), jnp.float32) for _ in range(NB)],
        [pltpu.SemaphoreType.DMA for _ in range(NB)],
        [pltpu.SemaphoreType.DMA for _ in range(NI)],
        [pltpu.SemaphoreType.DMA for _ in range(NI)],
        [pltpu.SemaphoreType.DMA for _ in range(NB)],
    ],
)
def _pool_kernel(src_hbm, dst_hbm, xs_hbm, out_hbm,
                 pooled_sh, srcb, dstb, rows, gsem, psem, dsem, ssem):
    c = lax.axis_index("c")
    s = lax.axis_index("s")
    zslice = NP // NS
    off = c * N

    def zfill(r, _):
        for j in range(HALF // L):
            rows[0][r, pl.ds(j * L, L)] = jnp.zeros((L,), jnp.float32)
        return 0

    lax.fori_loop(0, CH, zfill, 0)

    def zero(k, _):
        pltpu.sync_copy(rows[0],
                        pooled_sh.at[pl.ds(s * zslice + k * CH, CH), :])
        return 0

    lax.fori_loop(0, zslice // CH, zero, 0)
    plsc.subcore_barrier()

    def start_src(g, i):
        pltpu.async_copy(src_hbm.at[pl.ds(s * NCHT + g, 1), :], srcb[i],
                         psem[i])

    def wait_src(g, i):
        pltpu.make_async_copy(src_hbm.at[pl.ds(s * NCHT + g, 1), :],
                              srcb[i], psem[i]).wait()

    def start_dst(g, i):
        pltpu.async_copy(dst_hbm.at[pl.ds(s * NCHT + g, 1), :], dstb[i],
                         dsem[i])

    def wait_dst(g, i):
        pltpu.make_async_copy(dst_hbm.at[pl.ds(s * NCHT + g, 1), :],
                              dstb[i], dsem[i]).wait()

    def addoff(i):
        for j in range(CH // L):
            v = srcb[i][0, pl.ds(j * L, L)]
            srcb[i][0, pl.ds(j * L, L)] = v + off

    def start_gather(g, b, i):
        pltpu.async_copy(xs_hbm.at[srcb[i].at[0]], rows[b], gsem[b])

    def wait_gather(g, b, i):
        pltpu.make_async_copy(xs_hbm.at[srcb[i].at[0]], rows[b],
                              gsem[b]).wait()

    def start_scatter(g, b, i):
        pltpu.async_copy(rows[b], pooled_sh.at[dstb[i].at[0]], ssem[b],
                         add=True)

    def wait_scatter(g, b, i):
        pltpu.make_async_copy(rows[b], pooled_sh.at[dstb[i].at[0]],
                              ssem[b]).wait()

    for k in range(NI):
        start_src(k, k)
        start_dst(k, k)
    for b in range(NB):
        wait_src(b, b)
        addoff(b)
        start_gather(b, b, b)

    def body(k, _):
        for ii in range(NI):
            g = k * NI + ii
            bb = ii % NB
            wait_gather(g, bb, ii)
            wait_dst(g, ii)
            start_scatter(g, bb, ii)
            gi = g + NI
            gn = g + NB

            @pl.when(gn < NCHT)
            def _():
                wait_scatter(g, bb, ii)
                inew = (ii + NB) % NI

                @pl.when(gi < NCHT)
                def _():
                    start_src(gi, ii)
                    start_dst(gi, ii)

                wait_src(gn, inew)
                addoff(inew)
                start_gather(gn, bb, inew)
        return 0

    lax.fori_loop(0, NCHT // NI, body, 0)
    for bb in range(NB):
        g = NCHT - NB + bb
        wait_scatter(g, bb, g % NI)
    plsc.subcore_barrier()

    pltpu.sync_copy(
        pooled_sh.at[pl.ds(s * zslice, zslice), :],
        out_hbm.at[pl.ds(c * NP + s * zslice, zslice), :],
    )


def _dense_body(p_ref, isd_ref, w_ref, b_ref, o_ref):
    acc = jnp.dot(p_ref[0], w_ref[0], precision=lax.Precision.HIGHEST,
                  preferred_element_type=jnp.float32)
    acc += jnp.dot(p_ref[1], w_ref[1], precision=lax.Precision.HIGHEST,
                   preferred_element_type=jnp.float32)
    o_ref[...] = jnp.maximum(isd_ref[...] * acc + b_ref[...], 0.0)


def _dense_call(pooled3, isd, w3, b2):
    R = 2000
    grid = (N // R,)
    return pl.pallas_call(
        _dense_body,
        grid=grid,
        in_specs=[
            pl.BlockSpec((2, R, HALF), lambda i: (0, i, 0)),
            pl.BlockSpec((R, 1), lambda i: (i, 0)),
            pl.BlockSpec((2, HALF, U), lambda i: (0, 0, 0)),
            pl.BlockSpec((1, U), lambda i: (0, 0)),
        ],
        out_specs=pl.BlockSpec((R, U), lambda i: (i, 0)),
        out_shape=jax.ShapeDtypeStruct((N, U), jnp.float32),
    )(pooled3, isd, w3, b2)


@jax.jit
def kernel(x, edge_index, W, b):
    src = edge_index[0]
    dst = edge_index[1]
    pad = EPAD - E
    src_pad = jnp.concatenate([src, jnp.zeros((pad,), jnp.int32)])
    dst_pad = jnp.concatenate([dst, jnp.full((pad,), NP - 1, jnp.int32)])

    deg = _degree_kernel(dst_pad.reshape(EPAD // 128, 128))
    degc0 = deg[0, :N].reshape(N, 1)
    degc1 = deg[1, :N].reshape(N, 1)
    xs3, isd = _norm_call(x, degc0, degc1)
    pooled = _pool_kernel(src_pad.reshape(EPAD // CH, CH),
                          dst_pad.reshape(EPAD // CH, CH),
                          xs3.reshape(2 * N, HALF))
    pooled3 = pooled.reshape(NC, NP, HALF)
    out = _dense_call(pooled3, isd, W.reshape(2, HALF, U), b.reshape(1, U))
    return out
